# Initial kernel scaffold; baseline (speedup 1.0000x reference)
import jax
import jax.numpy as jnp
from jax.experimental import pallas as pl


def kernel(x, edge_index, a_w):
    raise NotImplementedError("write your pallas kernel here")


# speedup vs baseline: 20.3999x; 20.3999x over previous
import functools

import jax
import jax.numpy as jnp
from jax import lax
from jax.experimental import pallas as pl
from jax.experimental.pallas import tpu as pltpu
from jax.experimental.pallas import tpu_sc as plsc

N_NODES = 10000
N_EDGES = 320000
D = 128

CH = 80
NCHUNK = N_EDGES // CH
CPT = NCHUNK // 32
RPT = N_NODES // 16


def _scores_body(x_ref, a_ref, out_ref):
    out_ref[...] = jnp.dot(x_ref[...], a_ref[...],
                           preferred_element_type=jnp.float32)


def _node_scores(x, a2col):
    return pl.pallas_call(
        _scores_body,
        out_shape=jax.ShapeDtypeStruct((N_NODES, 2), jnp.float32),
    )(x, a2col)


def _gat_edges_sc(x, src2d, dst2d, s1, s2):
    mesh = plsc.VectorSubcoreMesh(core_axis_name="c", subcore_axis_name="s")

    @functools.partial(
        pl.kernel,
        out_type=(
            jax.ShapeDtypeStruct((2, N_NODES, D), jnp.float32),
            jax.ShapeDtypeStruct((2, 16, 640), jnp.float32),
        ),
        mesh=mesh,
        scratch_types=[
            pltpu.VMEM((CPT, CH), jnp.int32),
            pltpu.VMEM((CPT, CH), jnp.int32),
            pltpu.VMEM((CH,), jnp.float32),
            pltpu.VMEM((CH,), jnp.float32),
            pltpu.VMEM((CH,), jnp.float32),
            pltpu.VMEM((CH, D), jnp.float32),
            pltpu.VMEM((640,), jnp.float32),
            pltpu.VMEM_SHARED((N_NODES, D), jnp.float32),
            pltpu.VMEM_SHARED((16 * 640,), jnp.float32),
            pltpu.SemaphoreType.DMA,
            pltpu.SemaphoreType.DMA,
            pltpu.SemaphoreType.DMA,
            pltpu.SemaphoreType.DMA,
        ],
        compiler_params=pltpu.CompilerParams(needs_layout_passes=False),
    )
    def k(x_hbm, src_hbm, dst_hbm, s1_hbm, s2_hbm,
          hpart_hbm, dpart_hbm,
          src_i, dst_i, e1b, e2b, wbuf, rows, zd,
          h_sh, den_sh, gsem, ssem, sem1, sem2):
        cid = lax.axis_index("c")
        sid = lax.axis_index("s")

        zv = jnp.zeros((16,), jnp.float32)

        def _zb(i, carry):
            rows[i // 8, pl.ds((i % 8) * 16, 16)] = zv
            return carry
        lax.fori_loop(0, 640, _zb, 0)

        def _zd(i, carry):
            zd[pl.ds(i * 16, 16)] = zv
            return carry
        lax.fori_loop(0, 40, _zd, 0)

        row0 = sid * 640

        pltpu.sync_copy(zd, den_sh.at[pl.ds(row0, 640)])

        @pl.when(sid < 15)
        def _():
            for b in range(8):
                pltpu.sync_copy(rows, h_sh.at[pl.ds(row0 + b * 80, 80)])

        @pl.when(sid == 15)
        def _():
            for b in range(5):
                pltpu.sync_copy(rows, h_sh.at[pl.ds(9600 + b * 80, 80)])

        wid = cid * 16 + sid
        pltpu.sync_copy(src_hbm.at[wid], src_i)
        pltpu.sync_copy(dst_hbm.at[wid], dst_i)

        plsc.subcore_barrier()

        def _chunk(i, carry):
            src_row = src_i.at[i]
            dst_row = dst_i.at[i]
            gcp = pltpu.async_copy(x_hbm.at[src_row], rows, gsem)
            g1 = pltpu.async_copy(s1_hbm.at[src_row], e1b, sem1)
            g2 = pltpu.async_copy(s2_hbm.at[dst_row], e2b, sem2)
            g1.wait()
            g2.wait()
            for j in range(5):
                e = (e1b[pl.ds(j * 16, 16)] + e2b[pl.ds(j * 16, 16)])
                e = jnp.where(e >= 0.0, e, 0.01 * e)
                wbuf[pl.ds(j * 16, 16)] = jnp.exp(e)
            gcp.wait()

            def _scale(r, carry2):
                wb = plsc.load_gather(wbuf, [jnp.full((16,), r, jnp.int32)])
                for c in range(8):
                    rows[r, pl.ds(c * 16, 16)] = (
                        rows[r, pl.ds(c * 16, 16)] * wb)
                return carry2
            lax.fori_loop(0, CH, _scale, 0)

            pltpu.async_copy(rows, h_sh.at[dst_row], ssem, add=True).wait()
            pltpu.async_copy(wbuf, den_sh.at[dst_row], ssem, add=True).wait()
            return carry
        lax.fori_loop(0, CPT, _chunk, 0)

        plsc.subcore_barrier()

        pltpu.sync_copy(den_sh.at[pl.ds(row0, 640)],
                        dpart_hbm.at[cid, sid])

        @pl.when(sid < 15)
        def _():
            pltpu.sync_copy(h_sh.at[pl.ds(row0, 640)],
                            hpart_hbm.at[cid, pl.ds(row0, 640)])

        @pl.when(sid == 15)
        def _():
            pltpu.sync_copy(h_sh.at[pl.ds(9600, 400)],
                            hpart_hbm.at[cid, pl.ds(9600, 400)])

    return k(x, src2d, dst2d, s1, s2)


def _combine_body(h_ref, d_ref, out_ref):
    hs = h_ref[0] + h_ref[1]
    d = d_ref[0] + d_ref[1]
    d = jnp.where(d == 0.0, 1.0, d)
    out_ref[...] = jnp.maximum(hs / d[:, None], 0.0)


def _combine(h_part, den_part):
    return pl.pallas_call(
        _combine_body,
        out_shape=jax.ShapeDtypeStruct((N_NODES, D), jnp.float32),
    )(h_part, den_part)


def kernel(x, edge_index, a_w):
    a2col = a_w.reshape(2, D).T
    s12 = _node_scores(x, a2col)
    s1 = s12[:, 0] + 0.0
    s2 = s12[:, 1] + 0.0
    ei = edge_index.astype(jnp.int32)
    src2d = ei[0].reshape(32, CPT, CH)
    dst2d = ei[1].reshape(32, CPT, CH)
    h_part, den_part = _gat_edges_sc(x, src2d, dst2d, s1, s2)
    den = den_part.reshape(2, 16 * 640)[:, :N_NODES]
    return _combine(h_part, den)


# speedup vs baseline: 24.3777x; 1.1950x over previous
import functools

import jax
import jax.numpy as jnp
from jax import lax
from jax.experimental import pallas as pl
from jax.experimental.pallas import tpu as pltpu
from jax.experimental.pallas import tpu_sc as plsc

N_NODES = 10000
N_EDGES = 320000
D = 128

CH = 112
CPT = 90
NSLOT = 32 * CPT * CH
NACC = 16 * 640
DUMMY = N_NODES


def _scores_body(x_ref, a_ref, out_ref):
    out_ref[...] = jnp.dot(x_ref[...], a_ref[...],
                           preferred_element_type=jnp.float32)


def _node_scores(x, a2col):
    return pl.pallas_call(
        _scores_body,
        out_shape=jax.ShapeDtypeStruct((N_NODES, 2), jnp.float32),
    )(x, a2col)


def _gat_edges_sc(x, src3d, dst3d, s1, s2):
    mesh = plsc.VectorSubcoreMesh(core_axis_name="c", subcore_axis_name="s")

    @functools.partial(
        pl.kernel,
        out_type=(
            jax.ShapeDtypeStruct((2, N_NODES, D), jnp.float32),
            jax.ShapeDtypeStruct((2, 16, 640), jnp.float32),
        ),
        mesh=mesh,
        scratch_types=[
            [pltpu.VMEM((CH,), jnp.int32) for _ in range(3)],
            [pltpu.VMEM((CH,), jnp.int32) for _ in range(3)],
            [pltpu.VMEM((CH,), jnp.int32) for _ in range(3)],
            [pltpu.VMEM((CH,), jnp.float32) for _ in range(3)],
            [pltpu.VMEM((CH,), jnp.float32) for _ in range(3)],
            [pltpu.VMEM((CH,), jnp.float32) for _ in range(3)],
            [pltpu.VMEM((CH, D), jnp.float32) for _ in range(3)],
            pltpu.VMEM((640,), jnp.float32),
            pltpu.VMEM_SHARED((NACC, D), jnp.float32),
            pltpu.VMEM_SHARED((NACC,), jnp.float32),
            [pltpu.SemaphoreType.DMA for _ in range(3)],
            [pltpu.SemaphoreType.DMA for _ in range(3)],
            [pltpu.SemaphoreType.DMA for _ in range(3)],
            [pltpu.SemaphoreType.DMA for _ in range(3)],
            [pltpu.SemaphoreType.DMA for _ in range(3)],
            [pltpu.SemaphoreType.DMA for _ in range(3)],
            [pltpu.SemaphoreType.DMA for _ in range(3)],
        ],
        compiler_params=pltpu.CompilerParams(needs_layout_passes=False),
    )
    def k(x_hbm, src_hbm, dst_hbm, s1_hbm, s2_hbm,
          hpart_hbm, dpart_hbm,
          src_i, dst_i, dst_s, e1b, e2b, wbuf, rows, zd,
          h_sh, den_sh,
          sis, sid_, sgr, se1, se2, ssr, ssd):
        cid = lax.axis_index("c")
        sid = lax.axis_index("s")
        wid = cid * 16 + sid

        zv = jnp.zeros((16,), jnp.float32)

        def _zb(i, carry):
            rows[0][i // 8, pl.ds((i % 8) * 16, 16)] = zv
            return carry
        lax.fori_loop(0, CH * 8, _zb, 0)

        def _zd(i, carry):
            zd[pl.ds(i * 16, 16)] = zv
            return carry
        lax.fori_loop(0, 40, _zd, 0)

        row0 = sid * 640
        pltpu.sync_copy(zd, den_sh.at[pl.ds(row0, 640)])
        for b in range(8):
            pltpu.sync_copy(rows[0].at[pl.ds(0, 80)],
                            h_sh.at[pl.ds(row0 + b * 80, 80)])

        plsc.subcore_barrier()

        def issue_idx(c, k):
            pltpu.async_copy(src_hbm.at[wid, c], src_i[k], sis[k])
            pltpu.async_copy(dst_hbm.at[wid, c], dst_i[k], sid_[k])

        def wait_idx(k):
            pltpu.make_async_copy(src_hbm.at[0, 0], src_i[k], sis[k]).wait()
            pltpu.make_async_copy(dst_hbm.at[0, 0], dst_i[k], sid_[k]).wait()

        def issue_gather(k):
            pltpu.async_copy(x_hbm.at[src_i[k]], rows[k], sgr[k])
            pltpu.async_copy(s1_hbm.at[src_i[k]], e1b[k], se1[k])
            pltpu.async_copy(s2_hbm.at[dst_i[k]], e2b[k], se2[k])

        def wait_gather(k):
            pltpu.make_async_copy(s1_hbm.at[src_i[k]], e1b[k], se1[k]).wait()
            pltpu.make_async_copy(s2_hbm.at[dst_i[k]], e2b[k], se2[k]).wait()
            pltpu.make_async_copy(x_hbm.at[src_i[k]], rows[k], sgr[k]).wait()

        def issue_scatter(k):
            pltpu.async_copy(rows[k], h_sh.at[dst_s[k]], ssr[k], add=True)
            pltpu.async_copy(wbuf[k], den_sh.at[dst_s[k]], ssd[k], add=True)

        def wait_scatter(k):
            pltpu.make_async_copy(rows[k], h_sh.at[dst_s[k]], ssr[k]).wait()
            pltpu.make_async_copy(wbuf[k], den_sh.at[dst_s[k]], ssd[k]).wait()

        def compute(k):
            for j in range(CH // 16):
                dst_s[k][pl.ds(j * 16, 16)] = dst_i[k][pl.ds(j * 16, 16)]
                e = (e1b[k][pl.ds(j * 16, 16)] + e2b[k][pl.ds(j * 16, 16)])
                e = jnp.where(e >= 0.0, e, 0.01 * e)
                wbuf[k][pl.ds(j * 16, 16)] = jnp.exp(e)

            def _scale(r, carry):
                wb = plsc.load_gather(wbuf[k],
                                      [jnp.full((16,), r, jnp.int32)])
                for cc in range(D // 16):
                    rows[k][r, pl.ds(cc * 16, 16)] = (
                        rows[k][r, pl.ds(cc * 16, 16)] * wb)
                return carry
            lax.fori_loop(0, CH, _scale, 0)

        issue_idx(0, 0)
        issue_idx(1, 1)
        wait_idx(0)
        issue_gather(0)

        def _iter(i, carry):
            for k in range(3):
                c = 3 * i + k
                kn = (k + 1) % 3
                kp = (k + 2) % 3

                @pl.when(c >= 2)
                def _():
                    wait_scatter(kn)

                @pl.when(c + 1 <= CPT - 1)
                def _():
                    wait_idx(kn)
                    issue_gather(kn)

                @pl.when(c + 2 <= CPT - 1)
                def _():
                    issue_idx(c + 2, kp)

                wait_gather(k)
                compute(k)
                issue_scatter(k)
            return carry
        lax.fori_loop(0, CPT // 3, _iter, 0)

        wait_scatter((CPT - 2) % 3)
        wait_scatter((CPT - 1) % 3)

        plsc.subcore_barrier()

        pltpu.sync_copy(den_sh.at[pl.ds(row0, 640)],
                        dpart_hbm.at[cid, sid])

        @pl.when(sid < 15)
        def _():
            pltpu.sync_copy(h_sh.at[pl.ds(row0, 640)],
                            hpart_hbm.at[cid, pl.ds(row0, 640)])

        @pl.when(sid == 15)
        def _():
            pltpu.sync_copy(h_sh.at[pl.ds(9600, 400)],
                            hpart_hbm.at[cid, pl.ds(9600, 400)])

    return k(x, src3d, dst3d, s1, s2)


def _combine_body(h_ref, d_ref, out_ref):
    hs = h_ref[0] + h_ref[1]
    d = d_ref[0] + d_ref[1]
    d = jnp.where(d == 0.0, 1.0, d)
    out_ref[...] = jnp.maximum(hs / d[:, None], 0.0)


def _combine(h_part, den_part):
    return pl.pallas_call(
        _combine_body,
        out_shape=jax.ShapeDtypeStruct((N_NODES, D), jnp.float32),
    )(h_part, den_part)


def _prep_idx(edge_index):
    ei = edge_index.astype(jnp.int32)
    pad = NSLOT - N_EDGES
    src = jnp.pad(ei[0], (0, pad))
    dst = jnp.pad(ei[1], (0, pad), constant_values=DUMMY)
    return src.reshape(32, CPT, CH), dst.reshape(32, CPT, CH)


def kernel(x, edge_index, a_w):
    a2col = a_w.reshape(2, D).T
    s12 = _node_scores(x, a2col)
    s1 = s12[:, 0] + 0.0
    s2 = jnp.pad(s12[:, 1], (0, NACC - N_NODES))
    src3d, dst3d = _prep_idx(edge_index)
    h_part, den_part = _gat_edges_sc(x, src3d, dst3d, s1, s2)
    den = den_part.reshape(2, NACC)[:, :N_NODES]
    return _combine(h_part, den)


# speedup vs baseline: 24.7074x; 1.0135x over previous
import functools

import jax
import jax.numpy as jnp
from jax import lax
from jax.experimental import pallas as pl
from jax.experimental.pallas import tpu as pltpu
from jax.experimental.pallas import tpu_sc as plsc

N_NODES = 10000
N_EDGES = 320000
D = 128

CH = 112
CPT = 90
NSLOT = 32 * CPT * CH
NACC = 16 * 640
DUMMY = N_NODES


def _scores_body(x_ref, a_ref, out_ref):
    out_ref[...] = jnp.dot(x_ref[...], a_ref[...],
                           preferred_element_type=jnp.float32)


def _node_scores(x, a2col):
    return pl.pallas_call(
        _scores_body,
        out_shape=jax.ShapeDtypeStruct((N_NODES, 2), jnp.float32),
    )(x, a2col)


def _gat_edges_sc(x, src3d, dst3d, s1, s2):
    mesh = plsc.VectorSubcoreMesh(core_axis_name="c", subcore_axis_name="s")

    @functools.partial(
        pl.kernel,
        out_type=(
            jax.ShapeDtypeStruct((2, N_NODES, D), jnp.float32),
            jax.ShapeDtypeStruct((2, 16, 640), jnp.float32),
        ),
        mesh=mesh,
        scratch_types=[
            [pltpu.VMEM((CH,), jnp.int32) for _ in range(3)],
            [pltpu.VMEM((CH,), jnp.int32) for _ in range(3)],
            [pltpu.VMEM((CH,), jnp.int32) for _ in range(3)],
            [pltpu.VMEM((CH,), jnp.float32) for _ in range(3)],
            [pltpu.VMEM((CH,), jnp.float32) for _ in range(3)],
            [pltpu.VMEM((CH,), jnp.float32) for _ in range(3)],
            [pltpu.VMEM((CH, D), jnp.float32) for _ in range(3)],
            pltpu.VMEM((640,), jnp.float32),
            pltpu.VMEM_SHARED((NACC, D), jnp.float32),
            pltpu.VMEM_SHARED((NACC,), jnp.float32),
            [pltpu.SemaphoreType.DMA for _ in range(3)],
            [pltpu.SemaphoreType.DMA for _ in range(3)],
            [pltpu.SemaphoreType.DMA for _ in range(3)],
            [pltpu.SemaphoreType.DMA for _ in range(3)],
            [pltpu.SemaphoreType.DMA for _ in range(3)],
            [pltpu.SemaphoreType.DMA for _ in range(3)],
            [pltpu.SemaphoreType.DMA for _ in range(3)],
        ],
        compiler_params=pltpu.CompilerParams(needs_layout_passes=False),
    )
    def k(x_hbm, src_hbm, dst_hbm, s1_hbm, s2_hbm,
          hpart_hbm, dpart_hbm,
          src_i, dst_i, dst_s, e1b, e2b, wbuf, rows, zd,
          h_sh, den_sh,
          sis, sid_, sgr, se1, se2, ssr, ssd):
        cid = lax.axis_index("c")
        sid = lax.axis_index("s")
        wid = cid * 16 + sid

        zv = jnp.zeros((16,), jnp.float32)

        def _zb(i, carry):
            rows[0][i // 8, pl.ds((i % 8) * 16, 16)] = zv
            return carry
        lax.fori_loop(0, CH * 8, _zb, 0)

        def _zd(i, carry):
            zd[pl.ds(i * 16, 16)] = zv
            return carry
        lax.fori_loop(0, 40, _zd, 0)

        row0 = sid * 640
        pltpu.sync_copy(zd, den_sh.at[pl.ds(row0, 640)])
        for b in range(8):
            pltpu.sync_copy(rows[0].at[pl.ds(0, 80)],
                            h_sh.at[pl.ds(row0 + b * 80, 80)])

        plsc.subcore_barrier()

        def issue_idx(c, k):
            pltpu.async_copy(src_hbm.at[wid, c], src_i[k], sis[k])
            pltpu.async_copy(dst_hbm.at[wid, c], dst_i[k], sid_[k])

        def wait_idx(k):
            pltpu.make_async_copy(src_hbm.at[0, 0], src_i[k], sis[k]).wait()
            pltpu.make_async_copy(dst_hbm.at[0, 0], dst_i[k], sid_[k]).wait()

        def issue_gather(k):
            pltpu.async_copy(x_hbm.at[src_i[k]], rows[k], sgr[k])
            pltpu.async_copy(s1_hbm.at[src_i[k]], e1b[k], se1[k])
            pltpu.async_copy(s2_hbm.at[dst_i[k]], e2b[k], se2[k])

        def wait_gather(k):
            pltpu.make_async_copy(s1_hbm.at[src_i[k]], e1b[k], se1[k]).wait()
            pltpu.make_async_copy(s2_hbm.at[dst_i[k]], e2b[k], se2[k]).wait()
            pltpu.make_async_copy(x_hbm.at[src_i[k]], rows[k], sgr[k]).wait()

        def issue_scatter(k):
            pltpu.async_copy(rows[k], h_sh.at[dst_s[k]], ssr[k], add=True)
            pltpu.async_copy(wbuf[k], den_sh.at[dst_s[k]], ssd[k], add=True)

        def wait_scatter(k):
            pltpu.make_async_copy(rows[k], h_sh.at[dst_s[k]], ssr[k]).wait()
            pltpu.make_async_copy(wbuf[k], den_sh.at[dst_s[k]], ssd[k]).wait()

        def compute(k):
            for j in range(CH // 16):
                dst_s[k][pl.ds(j * 16, 16)] = dst_i[k][pl.ds(j * 16, 16)]
                e = (e1b[k][pl.ds(j * 16, 16)] + e2b[k][pl.ds(j * 16, 16)])
                e = jnp.where(e >= 0.0, e, 0.01 * e)
                wbuf[k][pl.ds(j * 16, 16)] = jnp.exp(e)

            @plsc.parallel_loop(0, CH, 1, unroll=4)
            def _scale(r):
                wb = plsc.load_gather(wbuf[k],
                                      [jnp.full((16,), r, jnp.int32)])
                for cc in range(D // 16):
                    rows[k][r, pl.ds(cc * 16, 16)] = (
                        rows[k][r, pl.ds(cc * 16, 16)] * wb)

        issue_idx(0, 0)
        issue_idx(1, 1)
        wait_idx(0)
        issue_gather(0)

        def _iter(i, carry):
            for k in range(3):
                c = 3 * i + k
                kn = (k + 1) % 3
                kp = (k + 2) % 3

                @pl.when(c >= 2)
                def _():
                    wait_scatter(kn)

                @pl.when(c + 1 <= CPT - 1)
                def _():
                    wait_idx(kn)
                    issue_gather(kn)

                @pl.when(c + 2 <= CPT - 1)
                def _():
                    issue_idx(c + 2, kp)

                wait_gather(k)
                compute(k)
                issue_scatter(k)
            return carry
        lax.fori_loop(0, CPT // 3, _iter, 0)

        wait_scatter((CPT - 2) % 3)
        wait_scatter((CPT - 1) % 3)

        plsc.subcore_barrier()

        pltpu.sync_copy(den_sh.at[pl.ds(row0, 640)],
                        dpart_hbm.at[cid, sid])

        @pl.when(sid < 15)
        def _():
            pltpu.sync_copy(h_sh.at[pl.ds(row0, 640)],
                            hpart_hbm.at[cid, pl.ds(row0, 640)])

        @pl.when(sid == 15)
        def _():
            pltpu.sync_copy(h_sh.at[pl.ds(9600, 400)],
                            hpart_hbm.at[cid, pl.ds(9600, 400)])

    return k(x, src3d, dst3d, s1, s2)


def _combine_body(h_ref, d_ref, out_ref):
    hs = h_ref[0] + h_ref[1]
    d = d_ref[0] + d_ref[1]
    d = jnp.where(d == 0.0, 1.0, d)
    out_ref[...] = jnp.maximum(hs / d[:, None], 0.0)


def _combine(h_part, den_part):
    return pl.pallas_call(
        _combine_body,
        out_shape=jax.ShapeDtypeStruct((N_NODES, D), jnp.float32),
    )(h_part, den_part)


def _prep_idx(edge_index):
    ei = edge_index.astype(jnp.int32)
    pad = NSLOT - N_EDGES
    src = jnp.pad(ei[0], (0, pad))
    dst = jnp.pad(ei[1], (0, pad), constant_values=DUMMY)
    return src.reshape(32, CPT, CH), dst.reshape(32, CPT, CH)


def kernel(x, edge_index, a_w):
    a2col = a_w.reshape(2, D).T
    s12 = _node_scores(x, a2col)
    s1 = s12[:, 0] + 0.0
    s2 = jnp.pad(s12[:, 1], (0, NACC - N_NODES))
    src3d, dst3d = _prep_idx(edge_index)
    h_part, den_part = _gat_edges_sc(x, src3d, dst3d, s1, s2)
    den = den_part.reshape(2, NACC)[:, :N_NODES]
    return _combine(h_part, den)


# speedup vs baseline: 24.9179x; 1.0085x over previous
import functools

import jax
import jax.numpy as jnp
from jax import lax
from jax.experimental import pallas as pl
from jax.experimental.pallas import tpu as pltpu
from jax.experimental.pallas import tpu_sc as plsc

N_NODES = 10000
N_EDGES = 320000
D = 128

CH = 112
CPT = 90
NSLOT = 32 * CPT * CH
NACC = 16 * 640
DUMMY = N_NODES


def _scores_body(x_ref, a_ref, out_ref):
    out_ref[...] = jnp.dot(x_ref[...], a_ref[...],
                           preferred_element_type=jnp.float32)


def _node_scores(x, a2col):
    return pl.pallas_call(
        _scores_body,
        out_shape=jax.ShapeDtypeStruct((N_NODES, 2), jnp.float32),
    )(x, a2col)


def _gat_edges_sc(x, src3d, dst3d, s1, s2):
    mesh = plsc.VectorSubcoreMesh(core_axis_name="c", subcore_axis_name="s")

    @functools.partial(
        pl.kernel,
        out_type=(
            jax.ShapeDtypeStruct((2, N_NODES, D), jnp.float32),
            jax.ShapeDtypeStruct((2, 16, 640), jnp.float32),
        ),
        mesh=mesh,
        scratch_types=[
            [pltpu.VMEM((CH,), jnp.int32) for _ in range(3)],
            [pltpu.VMEM((CH,), jnp.int32) for _ in range(3)],
            [pltpu.VMEM((CH,), jnp.int32) for _ in range(3)],
            [pltpu.VMEM((CH,), jnp.float32) for _ in range(3)],
            [pltpu.VMEM((CH,), jnp.float32) for _ in range(3)],
            [pltpu.VMEM((CH,), jnp.float32) for _ in range(3)],
            [pltpu.VMEM((CH, D), jnp.float32) for _ in range(3)],
            pltpu.VMEM((640,), jnp.float32),
            pltpu.VMEM_SHARED((NACC, D), jnp.float32),
            pltpu.VMEM_SHARED((NACC,), jnp.float32),
            [pltpu.SemaphoreType.DMA for _ in range(3)],
            [pltpu.SemaphoreType.DMA for _ in range(3)],
            [pltpu.SemaphoreType.DMA for _ in range(3)],
            [pltpu.SemaphoreType.DMA for _ in range(3)],
            [pltpu.SemaphoreType.DMA for _ in range(3)],
            [pltpu.SemaphoreType.DMA for _ in range(3)],
            [pltpu.SemaphoreType.DMA for _ in range(3)],
        ],
        compiler_params=pltpu.CompilerParams(needs_layout_passes=False),
    )
    def k(x_hbm, src_hbm, dst_hbm, s1_hbm, s2_hbm,
          hpart_hbm, dpart_hbm,
          src_i, dst_i, dst_s, e1b, e2b, wbuf, rows, zd,
          h_sh, den_sh,
          sis, sid_, sgr, se1, se2, ssr, ssd):
        cid = lax.axis_index("c")
        sid = lax.axis_index("s")
        wid = cid * 16 + sid

        row0 = sid * 640

        def issue_idx(c, k):
            pltpu.async_copy(src_hbm.at[wid, c], src_i[k], sis[k])
            pltpu.async_copy(dst_hbm.at[wid, c], dst_i[k], sid_[k])

        def wait_idx(k):
            pltpu.make_async_copy(src_hbm.at[0, 0], src_i[k], sis[k]).wait()
            pltpu.make_async_copy(dst_hbm.at[0, 0], dst_i[k], sid_[k]).wait()

        def issue_gather(k):
            pltpu.async_copy(x_hbm.at[src_i[k]], rows[k], sgr[k])
            pltpu.async_copy(s1_hbm.at[src_i[k]], e1b[k], se1[k])
            pltpu.async_copy(s2_hbm.at[dst_i[k]], e2b[k], se2[k])

        def wait_gather(k):
            pltpu.make_async_copy(s1_hbm.at[src_i[k]], e1b[k], se1[k]).wait()
            pltpu.make_async_copy(s2_hbm.at[dst_i[k]], e2b[k], se2[k]).wait()
            pltpu.make_async_copy(x_hbm.at[src_i[k]], rows[k], sgr[k]).wait()

        def issue_scatter(k):
            pltpu.async_copy(rows[k], h_sh.at[dst_s[k]], ssr[k], add=True)
            pltpu.async_copy(wbuf[k], den_sh.at[dst_s[k]], ssd[k], add=True)

        def wait_scatter(k):
            pltpu.make_async_copy(rows[k], h_sh.at[dst_s[k]], ssr[k]).wait()
            pltpu.make_async_copy(wbuf[k], den_sh.at[dst_s[k]], ssd[k]).wait()

        def compute(k):
            for j in range(CH // 16):
                dst_s[k][pl.ds(j * 16, 16)] = dst_i[k][pl.ds(j * 16, 16)]
                e = (e1b[k][pl.ds(j * 16, 16)] + e2b[k][pl.ds(j * 16, 16)])
                e = jnp.where(e >= 0.0, e, 0.01 * e)
                wbuf[k][pl.ds(j * 16, 16)] = jnp.exp(e)

            @plsc.parallel_loop(0, CH, 1, unroll=4)
            def _scale(r):
                wb = plsc.load_gather(wbuf[k],
                                      [jnp.full((16,), r, jnp.int32)])
                for cc in range(D // 16):
                    rows[k][r, pl.ds(cc * 16, 16)] = (
                        rows[k][r, pl.ds(cc * 16, 16)] * wb)

        issue_idx(0, 0)
        issue_idx(1, 1)
        wait_idx(0)
        issue_gather(0)

        zv = jnp.zeros((16,), jnp.float32)

        def _zb(i, carry):
            rows[2][i // 8, pl.ds((i % 8) * 16, 16)] = zv
            return carry
        lax.fori_loop(0, 640, _zb, 0)

        def _zd(i, carry):
            zd[pl.ds(i * 16, 16)] = zv
            return carry
        lax.fori_loop(0, 40, _zd, 0)

        pltpu.sync_copy(zd, den_sh.at[pl.ds(row0, 640)])
        for b in range(8):
            pltpu.sync_copy(rows[2].at[pl.ds(0, 80)],
                            h_sh.at[pl.ds(row0 + b * 80, 80)])

        plsc.subcore_barrier()

        def _iter(i, carry):
            for k in range(3):
                c = 3 * i + k
                kn = (k + 1) % 3
                kp = (k + 2) % 3

                @pl.when(c >= 2)
                def _():
                    wait_scatter(kn)

                @pl.when(c + 1 <= CPT - 1)
                def _():
                    wait_idx(kn)
                    issue_gather(kn)

                @pl.when(c + 2 <= CPT - 1)
                def _():
                    issue_idx(c + 2, kp)

                wait_gather(k)
                compute(k)
                issue_scatter(k)
            return carry
        lax.fori_loop(0, CPT // 3, _iter, 0)

        wait_scatter((CPT - 2) % 3)
        wait_scatter((CPT - 1) % 3)

        plsc.subcore_barrier()

        pltpu.sync_copy(den_sh.at[pl.ds(row0, 640)],
                        dpart_hbm.at[cid, sid])

        @pl.when(sid < 15)
        def _():
            pltpu.sync_copy(h_sh.at[pl.ds(row0, 640)],
                            hpart_hbm.at[cid, pl.ds(row0, 640)])

        @pl.when(sid == 15)
        def _():
            pltpu.sync_copy(h_sh.at[pl.ds(9600, 400)],
                            hpart_hbm.at[cid, pl.ds(9600, 400)])

    return k(x, src3d, dst3d, s1, s2)


def _combine_body(h_ref, d_ref, out_ref):
    hs = h_ref[0] + h_ref[1]
    d = d_ref[0] + d_ref[1]
    d = jnp.where(d == 0.0, 1.0, d)
    out_ref[...] = jnp.maximum(hs / d[:, None], 0.0)


def _combine(h_part, den_part):
    return pl.pallas_call(
        _combine_body,
        out_shape=jax.ShapeDtypeStruct((N_NODES, D), jnp.float32),
    )(h_part, den_part)


def _prep_idx(edge_index):
    ei = edge_index.astype(jnp.int32)
    pad = NSLOT - N_EDGES
    src = jnp.pad(ei[0], (0, pad))
    dst = jnp.pad(ei[1], (0, pad), constant_values=DUMMY)
    return src.reshape(32, CPT, CH), dst.reshape(32, CPT, CH)


def kernel(x, edge_index, a_w):
    a2col = a_w.reshape(2, D).T
    s12 = _node_scores(x, a2col)
    s1 = s12[:, 0] + 0.0
    s2 = jnp.pad(s12[:, 1], (0, NACC - N_NODES))
    src3d, dst3d = _prep_idx(edge_index)
    h_part, den_part = _gat_edges_sc(x, src3d, dst3d, s1, s2)
    den = den_part.reshape(2, NACC)[:, :N_NODES]
    return _combine(h_part, den)
